# baseline (speedup 1.0000x reference)
import functools

import jax
import jax.numpy as jnp
from jax import lax
from jax.experimental import pallas as pl
from jax.experimental.pallas import tpu as pltpu
from jax.experimental.pallas import tpu_sc as plsc

VOCAB = 1000000
MAXLEN = 200
EMBED_DIM = 64
BATCH = 4096

NUM_CORES = 2
NUM_SUBCORES = 16
LANES = 16
NW = NUM_CORES * NUM_SUBCORES
ITEMS_PER_W = BATCH // NW
VECS_PER_ROW = EMBED_DIM // LANES


def _make_kernel():
    mesh = plsc.VectorSubcoreMesh(core_axis_name="c", subcore_axis_name="s")

    @functools.partial(
        pl.kernel,
        out_type=jax.ShapeDtypeStruct((BATCH, MAXLEN, EMBED_DIM), jnp.float32),
        mesh=mesh,
        scratch_types=[
            pltpu.VMEM((MAXLEN, EMBED_DIM), jnp.float32),
            pltpu.VMEM((MAXLEN,), jnp.int32),
            pltpu.VMEM((MAXLEN, EMBED_DIM), jnp.float32),
            pltpu.SemaphoreType.DMA,
        ],
        compiler_params=pltpu.CompilerParams(use_tc_tiling_on_sc=False),
    )
    def tok_pos_embed(x_hbm, tok_hbm, pos_hbm, out_hbm, pos_v, idx_v, rows_v, sem):
        wid = lax.axis_index("s") * NUM_CORES + lax.axis_index("c")
        base_item = wid * ITEMS_PER_W
        pltpu.sync_copy(pos_hbm, pos_v)

        def item_body(i, carry):
            b = base_item + i
            pltpu.sync_copy(x_hbm.at[b], idx_v)
            pltpu.async_copy(tok_hbm.at[idx_v], rows_v, sem).wait()

            def add_row(r, c2):
                for cpart in range(VECS_PER_ROW):
                    sl = pl.ds(cpart * LANES, LANES)
                    plsc.addupdate(rows_v.at[r, sl], pos_v[r, sl])
                return c2

            lax.fori_loop(0, MAXLEN, add_row, 0, unroll=2)
            pltpu.sync_copy(rows_v, out_hbm.at[b])
            return carry

        lax.fori_loop(0, ITEMS_PER_W, item_body, 0)

    return tok_pos_embed


_kernel_call = _make_kernel()


def kernel(x, token_table, pos_table):
    return _kernel_call(x.astype(jnp.int32), token_table, pos_table)
